# Initial kernel scaffold; baseline (speedup 1.0000x reference)
import jax
import jax.numpy as jnp
from jax.experimental import pallas as pl


def kernel(x, W_embed):
    raise NotImplementedError("write your pallas kernel here")


# speedup vs baseline: 1.8609x; 1.8609x over previous
import functools

import jax
import jax.numpy as jnp
from jax import lax
from jax.experimental import pallas as pl
from jax.experimental.pallas import tpu as pltpu
from jax.experimental.pallas import tpu_sc as plsc

_NUM_CORES = 2
_NUM_SUBCORES = 16
_NUM_WORKERS = _NUM_CORES * _NUM_SUBCORES
_NBUF = 2


@functools.partial(jax.jit, static_argnums=(2,))
def _gather_rows(table, idx, chunk):
    total = idx.shape[0]
    depth = table.shape[1]
    rows_per_worker = total // _NUM_WORKERS
    n_chunks = rows_per_worker // chunk
    assert rows_per_worker % chunk == 0 and chunk % 8 == 0

    mesh = plsc.VectorSubcoreMesh(
        core_axis_name="c",
        subcore_axis_name="s",
        num_cores=_NUM_CORES,
        num_subcores=_NUM_SUBCORES,
    )

    @functools.partial(
        pl.kernel,
        out_type=jax.ShapeDtypeStruct((total, depth), jnp.float32),
        mesh=mesh,
        compiler_params=pltpu.CompilerParams(use_tc_tiling_on_sc=False),
        scratch_types=[
            pltpu.VMEM((chunk,), jnp.int32),
            pltpu.VMEM((chunk,), jnp.int32),
            pltpu.VMEM((chunk, depth), jnp.float32),
            pltpu.VMEM((chunk, depth), jnp.float32),
            pltpu.SemaphoreType.DMA,
            pltpu.SemaphoreType.DMA,
        ],
    )
    def grab(table_hbm, idx_hbm, out_hbm, idx0, idx1, rows0, rows1, sem0, sem1):
        c = lax.axis_index("c")
        s = lax.axis_index("s")
        wid = s * _NUM_CORES + c
        base = wid * rows_per_worker
        idxs = [idx0, idx1]
        rows = [rows0, rows1]
        sems = [sem0, sem1]

        def gather_desc(slot):
            return pltpu.make_async_copy(
                table_hbm.at[idxs[slot]], rows[slot], sems[slot]
            )

        def start(slot, g):
            off = base + g * chunk
            pltpu.sync_copy(idx_hbm.at[pl.ds(off, chunk)], idxs[slot])
            gather_desc(slot).start()

        for b in range(_NBUF):
            start(b, b)

        def body(i, _):
            g0 = i * _NBUF
            for b in range(_NBUF):
                g = g0 + b
                gather_desc(b).wait()
                off = base + g * chunk
                pltpu.sync_copy(rows[b], out_hbm.at[pl.ds(off, chunk)])
                nxt = g + _NBUF

                @pl.when(nxt < n_chunks)
                def _():
                    start(b, nxt)
            return ()

        lax.fori_loop(0, n_chunks // _NBUF, body, (), unroll=False)

    return grab(table, idx)


def kernel(x, W_embed):
    batch, seq = x.shape
    depth = W_embed.shape[1]
    idx = x.reshape(batch * seq).astype(jnp.int32)
    out = _gather_rows(W_embed, idx, 800)
    return out.reshape(batch, seq, depth)


# speedup vs baseline: 1.8900x; 1.0156x over previous
import functools

import jax
import jax.numpy as jnp
from jax import lax
from jax.experimental import pallas as pl
from jax.experimental.pallas import tpu as pltpu
from jax.experimental.pallas import tpu_sc as plsc

_NUM_CORES = 2
_NUM_SUBCORES = 16
_NUM_WORKERS = _NUM_CORES * _NUM_SUBCORES
_NBUF = 4
_LEAD = 2


@functools.partial(jax.jit, static_argnums=(2,))
def _gather_rows(table, idx, chunk):
    total = idx.shape[0]
    depth = table.shape[1]
    rows_per_worker = total // _NUM_WORKERS
    n_chunks = rows_per_worker // chunk
    assert rows_per_worker % chunk == 0 and chunk % 8 == 0
    assert n_chunks % _NBUF == 0 and _NBUF > _LEAD >= 1

    mesh = plsc.VectorSubcoreMesh(
        core_axis_name="c",
        subcore_axis_name="s",
        num_cores=_NUM_CORES,
        num_subcores=_NUM_SUBCORES,
    )

    @functools.partial(
        pl.kernel,
        out_type=jax.ShapeDtypeStruct((total, depth), jnp.float32),
        mesh=mesh,
        compiler_params=pltpu.CompilerParams(use_tc_tiling_on_sc=False),
        scratch_types=[
            [pltpu.VMEM((chunk,), jnp.int32)] * _NBUF,
            [pltpu.VMEM((chunk, depth), jnp.float32)] * _NBUF,
            [pltpu.SemaphoreType.DMA] * _NBUF,
            [pltpu.SemaphoreType.DMA] * _NBUF,
        ],
    )
    def grab(table_hbm, idx_hbm, out_hbm, idxs, rows, gsems, osems):
        c = lax.axis_index("c")
        s = lax.axis_index("s")
        wid = s * _NUM_CORES + c
        base = wid * rows_per_worker

        def gather_desc(slot):
            return pltpu.make_async_copy(
                table_hbm.at[idxs[slot]], rows[slot], gsems[slot]
            )

        def write_desc(slot, g):
            off = base + g * chunk
            return pltpu.make_async_copy(
                rows[slot], out_hbm.at[pl.ds(off, chunk)], osems[slot]
            )

        def start_gather(slot, g):
            off = base + g * chunk
            pltpu.sync_copy(idx_hbm.at[pl.ds(off, chunk)], idxs[slot])
            gather_desc(slot).start()

        for b in range(_LEAD):
            start_gather(b, b)

        def body(i, _):
            g0 = i * _NBUF
            for b in range(_NBUF):
                g = g0 + b
                gather_desc(b).wait()
                write_desc(b, g).start()
                bn = (b + _LEAD) % _NBUF
                nxt = g + _LEAD

                @pl.when(nxt < n_chunks)
                def _():
                    @pl.when(nxt >= _NBUF)
                    def _():
                        write_desc(bn, g).wait()

                    start_gather(bn, nxt)
            return ()

        lax.fori_loop(0, n_chunks // _NBUF, body, (), unroll=False)

        for b in range(_NBUF):
            write_desc(b, 0).wait()

    return grab(table, idx)


def kernel(x, W_embed):
    batch, seq = x.shape
    depth = W_embed.shape[1]
    idx = x.reshape(batch * seq).astype(jnp.int32)
    out = _gather_rows(W_embed, idx, 400)
    return out.reshape(batch, seq, depth)
